# Initial kernel scaffold; baseline (speedup 1.0000x reference)
import jax
import jax.numpy as jnp
from jax.experimental import pallas as pl
from jax.experimental.pallas import tpu as pltpu


def _dot_t(a, b):
    return jax.lax.dot_general(a, b, (((1,), (1,)), ((), ())),
                               preferred_element_type=jnp.float32)


def _dot(a, b):
    return jax.lax.dot_general(a, b, (((1,), (0,)), ((), ())),
                               preferred_element_type=jnp.float32)


def _proj_kernel(dt_ref, mask_ref, rl_ref, base_ref,
                 ce_ref, re_ref, dwrow_ref, db_ref,
                 wb_ref, wc_ref, wr_ref, wd_ref, pb_ref,
                 out_ref):
    G, Nn, Dd = base_ref.shape
    R = G * Nn
    n_run = re_ref.shape[0]

    cc = _dot_t(ce_ref[...], wc_ref[...])
    rr = _dot_t(re_ref[...], wr_ref[...])
    vv = _dot_t(dwrow_ref[...], wd_ref[...])
    const = _dot_t(db_ref[...], wd_ref[...]) + pb_ref[...]

    x = base_ref[...].reshape(R, Dd)
    acc = _dot_t(x, wb_ref[...])

    rl = jnp.clip(rl_ref[...], 0, n_run - 1).reshape(R, 1)
    oh = (jax.lax.broadcasted_iota(jnp.int32, (R, n_run), 1) == rl)
    acc = acc + _dot(oh.astype(jnp.float32), rr)

    maskf = mask_ref[...].reshape(R, 1)
    acc = acc + cc[0:1, :] + maskf * (cc[1:2, :] - cc[0:1, :]) + const

    dterm = dt_ref[...].reshape(G, 1, 1) * vv.reshape(1, 1, Dd)
    out_ref[...] = acc.reshape(G, Nn, Dd) + dterm


def kernel(base, change_mask, run_length, delta_t, change_embed, run_embed,
           delta_W, delta_b, proj_W, proj_b):
    Bb, Tt, Nn, Dd = base.shape
    Ee = change_embed.shape[1]
    BT = Bb * Tt
    G = 8
    grid = (BT // G,)

    base2 = base.reshape(BT, Nn, Dd)
    mask2 = change_mask.reshape(BT, Nn).astype(jnp.float32)
    rl2 = run_length.reshape(BT, Nn).astype(jnp.int32)
    dt2 = delta_t.astype(jnp.float32).reshape(BT, 1)
    wb = proj_W[:, :Dd]
    wc = proj_W[:, Dd:Dd + Ee]
    wr = proj_W[:, Dd + Ee:Dd + 2 * Ee]
    wd = proj_W[:, Dd + 2 * Ee:Dd + 3 * Ee]
    dwrow = delta_W.reshape(1, Ee)
    db2 = delta_b.reshape(1, Ee)
    pb2 = proj_b.reshape(1, Dd)

    rep = lambda shape: pl.BlockSpec(shape, lambda i: (0, 0))
    out = pl.pallas_call(
        _proj_kernel,
        grid=grid,
        in_specs=[
            pl.BlockSpec((G, 1), lambda i: (i, 0)),
            pl.BlockSpec((G, Nn), lambda i: (i, 0)),
            pl.BlockSpec((G, Nn), lambda i: (i, 0)),
            pl.BlockSpec((G, Nn, Dd), lambda i: (i, 0, 0)),
            rep(change_embed.shape),
            rep(run_embed.shape),
            rep((1, Ee)),
            rep((1, Ee)),
            rep((Dd, Dd)),
            rep((Dd, Ee)),
            rep((Dd, Ee)),
            rep((Dd, Ee)),
            rep((1, Dd)),
        ],
        out_specs=pl.BlockSpec((G, Nn, Dd), lambda i: (i, 0, 0)),
        out_shape=jax.ShapeDtypeStruct((BT, Nn, Dd), jnp.float32),
        compiler_params=pltpu.CompilerParams(
            dimension_semantics=("parallel",)),
    )(dt2, mask2, rl2, base2, change_embed, run_embed, dwrow, db2,
      wb, wc, wr, wd, pb2)
    return out.reshape(Bb, Tt, Nn, Dd)


# speedup vs baseline: 7.2299x; 7.2299x over previous
import jax
import jax.numpy as jnp
from jax.experimental import pallas as pl
from jax.experimental.pallas import tpu as pltpu


def _dot_t(a, b):
    return jax.lax.dot_general(a, b, (((1,), (1,)), ((), ())),
                               preferred_element_type=jnp.float32)


def _dot_kk(a, b):
    return jax.lax.dot_general(a, b, (((0,), (0,)), ((), ())),
                               preferred_element_type=jnp.float32)


def _proj_kernel(dt_ref, maskf_ref, rl_ref, base_ref,
                 ce_ref, re_ref, dwrow_ref, db_ref,
                 wb_ref, wc_ref, wr_ref, wd_ref, pb_ref,
                 out_ref):
    G, Nn, Dd = base_ref.shape
    R = G * Nn
    n_run = re_ref.shape[0]
    n_tab = 2 * n_run

    cc = _dot_t(ce_ref[...], wc_ref[...])
    rr = _dot_t(re_ref[...], wr_ref[...])
    vv = _dot_t(dwrow_ref[...], wd_ref[...])
    const = _dot_t(db_ref[...], wd_ref[...]) + pb_ref[...]
    table = jnp.concatenate([rr + cc[0:1, :] + const,
                             rr + cc[1:2, :] + const], axis=0)

    x = base_ref[...].reshape(R, Dd)
    mm = _dot_t(x, wb_ref[...]).reshape(G, Nn, Dd)

    idxf = (maskf_ref[...] * n_run
            + jnp.clip(rl_ref[...], 0, n_run - 1).astype(jnp.float32))

    kio = jax.lax.broadcasted_iota(jnp.int32, (n_tab, Nn), 0).astype(
        jnp.float32)
    for g in range(G):
        idx_row = idxf[g:g + 1, :]
        oh_t = (kio == idx_row).astype(jnp.float32)
        lk = _dot_kk(oh_t, table)
        out_ref[g, :, :] = mm[g] + lk + dt_ref[g, 0] * vv


def kernel(base, change_mask, run_length, delta_t, change_embed, run_embed,
           delta_W, delta_b, proj_W, proj_b):
    Bb, Tt, Nn, Dd = base.shape
    Ee = change_embed.shape[1]
    n_run = run_embed.shape[0]
    BT = Bb * Tt
    G = 8
    grid = (BT // G,)

    base2 = base.reshape(BT, Nn, Dd)
    mask2 = change_mask.reshape(BT, Nn).astype(jnp.float32)
    rl2 = run_length.reshape(BT, Nn).astype(jnp.int32)
    dt2 = delta_t.astype(jnp.float32).reshape(BT, 1)
    wb = proj_W[:, :Dd]
    wc = proj_W[:, Dd:Dd + Ee]
    wr = proj_W[:, Dd + Ee:Dd + 2 * Ee]
    wd = proj_W[:, Dd + 2 * Ee:Dd + 3 * Ee]
    dwrow = delta_W.reshape(1, Ee)
    db2 = delta_b.reshape(1, Ee)
    pb2 = proj_b.reshape(1, Dd)

    rep = lambda shape: pl.BlockSpec(shape, lambda i: (0, 0))
    out = pl.pallas_call(
        _proj_kernel,
        grid=grid,
        in_specs=[
            pl.BlockSpec((G, 1), lambda i: (i, 0)),
            pl.BlockSpec((G, Nn), lambda i: (i, 0)),
            pl.BlockSpec((G, Nn), lambda i: (i, 0)),
            pl.BlockSpec((G, Nn, Dd), lambda i: (i, 0, 0)),
            rep(change_embed.shape),
            rep(run_embed.shape),
            rep((1, Ee)),
            rep((1, Ee)),
            rep((Dd, Dd)),
            rep((Dd, Ee)),
            rep((Dd, Ee)),
            rep((Dd, Ee)),
            rep((1, Dd)),
        ],
        out_specs=pl.BlockSpec((G, Nn, Dd), lambda i: (i, 0, 0)),
        out_shape=jax.ShapeDtypeStruct((BT, Nn, Dd), jnp.float32),
        compiler_params=pltpu.CompilerParams(
            dimension_semantics=("parallel",)),
    )(dt2, mask2, rl2, base2, change_embed, run_embed, dwrow, db2,
      wb, wc, wr, wd, pb2)
    return out.reshape(Bb, Tt, Nn, Dd)
